# baseline (speedup 1.0000x reference)
import jax
import jax.numpy as jnp
from jax.experimental import pallas as pl

B, S, H = 4, 128, 768
NEG_SLOPE = 0.2


def _gat_kernel(x_ref, t_ref, wg_ref, a2_ref, bg_ref, wf1_ref, wf2_ref,
                bf_ref, out_ref):
    bf16 = jnp.bfloat16
    x = x_ref[...]
    t = t_ref[...]

    xp = jnp.dot(x.astype(bf16), wg_ref[...].astype(bf16),
                 preferred_element_type=jnp.float32)
    sa = jnp.dot(xp.astype(bf16), a2_ref[...].astype(bf16),
                 preferred_element_type=jnp.float32)
    out_t = jnp.dot(t.astype(bf16), wf1_ref[...].astype(bf16),
                    preferred_element_type=jnp.float32)

    row = jax.lax.broadcasted_iota(jnp.int32, (S, S), 0)
    col = jax.lax.broadcasted_iota(jnp.int32, (S, S), 1)
    diag = row == col

    gs = []
    for b in range(B):
        sl = slice(b * S, (b + 1) * S)
        alpha_s = sa[sl, 0]
        alpha_d = sa[sl, 1]
        logits = alpha_d[:, None] + alpha_s[None, :]
        logits = jnp.where(logits > 0, logits, NEG_SLOPE * logits)
        logits = jnp.where(diag, -jnp.inf, logits)
        m = jnp.max(logits, axis=1, keepdims=True)
        ex = jnp.exp(logits - m)
        attn = ex / jnp.sum(ex, axis=1, keepdims=True)
        g = jnp.dot(attn.astype(bf16), xp[sl].astype(bf16),
                    preferred_element_type=jnp.float32)
        gs.append(g)

    g_all = jnp.concatenate(gs, axis=0) + bg_ref[...]
    out_g = jnp.dot(g_all.astype(bf16), wf2_ref[...].astype(bf16),
                    preferred_element_type=jnp.float32)
    out_ref[...] = out_t + out_g + bf_ref[...]


@jax.jit
def kernel(hidden_states, transformer_output, W_gat, a_src, a_dst, b_gat, W_fus, b_fus):
    x2 = hidden_states.reshape(B * S, H)
    t2 = transformer_output.reshape(B * S, H)
    a2 = jnp.stack([a_src, a_dst], axis=1)
    bg = b_gat.reshape(1, H)
    bf = b_fus.reshape(1, H)
    wf1 = W_fus[:H]
    wf2 = W_fus[H:]

    out = pl.pallas_call(
        _gat_kernel,
        out_shape=jax.ShapeDtypeStruct((B * S, H), jnp.float32),
    )(x2, t2, W_gat, a2, bg, wf1, wf2, bf)
    return out.reshape(B, S, H)


# speedup vs baseline: 1.0021x; 1.0021x over previous
import jax
import jax.numpy as jnp
from jax.experimental import pallas as pl

B, S, H = 4, 128, 768
NEG_SLOPE = 0.2
NB = 2
IB = B // NB
R = IB * S


def _gat_kernel(x_ref, t_ref, wg_ref, a2_ref, bg_ref, wf1_ref, wf2_ref,
                bf_ref, out_ref):
    bf16 = jnp.bfloat16
    x = x_ref[...].reshape(R, H)
    t = t_ref[...].reshape(R, H)

    xp = jnp.dot(x.astype(bf16), wg_ref[...].astype(bf16),
                 preferred_element_type=jnp.float32)
    sa = jnp.dot(xp.astype(bf16), a2_ref[...].astype(bf16),
                 preferred_element_type=jnp.float32)
    out_t = jnp.dot(t.astype(bf16), wf1_ref[...].astype(bf16),
                    preferred_element_type=jnp.float32)

    row = jax.lax.broadcasted_iota(jnp.int32, (S, S), 0)
    col = jax.lax.broadcasted_iota(jnp.int32, (S, S), 1)
    diag = row == col

    gs = []
    for b in range(IB):
        sl = slice(b * S, (b + 1) * S)
        alpha_s = sa[sl, 0]
        alpha_d = sa[sl, 1]
        logits = alpha_d[:, None] + alpha_s[None, :]
        logits = jnp.where(logits > 0, logits, NEG_SLOPE * logits)
        logits = jnp.where(diag, -jnp.inf, logits)
        m = jnp.max(logits, axis=1, keepdims=True)
        ex = jnp.exp(logits - m)
        attn = ex / jnp.sum(ex, axis=1, keepdims=True)
        g = jnp.dot(attn.astype(bf16), xp[sl].astype(bf16),
                    preferred_element_type=jnp.float32)
        gs.append(g)

    g_all = jnp.concatenate(gs, axis=0) + bg_ref[...]
    out_g = jnp.dot(g_all.astype(bf16), wf2_ref[...].astype(bf16),
                    preferred_element_type=jnp.float32)
    out_ref[...] = (out_t + out_g + bf_ref[...]).reshape(1, R, H)


@jax.jit
def kernel(hidden_states, transformer_output, W_gat, a_src, a_dst, b_gat, W_fus, b_fus):
    x3 = hidden_states.reshape(NB, R, H)
    t3 = transformer_output.reshape(NB, R, H)
    a2 = jnp.stack([a_src, a_dst], axis=1)
    bg = b_gat.reshape(1, H)
    bf = b_fus.reshape(1, H)
    wf1 = W_fus[:H]
    wf2 = W_fus[H:]

    out = pl.pallas_call(
        _gat_kernel,
        grid=(NB,),
        in_specs=[
            pl.BlockSpec((1, R, H), lambda i: (i, 0, 0)),
            pl.BlockSpec((1, R, H), lambda i: (i, 0, 0)),
            pl.BlockSpec((H, H), lambda i: (0, 0)),
            pl.BlockSpec((H, 2), lambda i: (0, 0)),
            pl.BlockSpec((1, H), lambda i: (0, 0)),
            pl.BlockSpec((H, H), lambda i: (0, 0)),
            pl.BlockSpec((H, H), lambda i: (0, 0)),
            pl.BlockSpec((1, H), lambda i: (0, 0)),
        ],
        out_specs=pl.BlockSpec((1, R, H), lambda i: (i, 0, 0)),
        out_shape=jax.ShapeDtypeStruct((NB, R, H), jnp.float32),
    )(x3, t3, W_gat, a2, bg, wf1, wf2, bf)
    return out.reshape(B, S, H)
